# Initial kernel scaffold; baseline (speedup 1.0000x reference)
import jax
import jax.numpy as jnp
from jax.experimental import pallas as pl


def kernel(indices, table):
    raise NotImplementedError("write your pallas kernel here")


# speedup vs baseline: 3.3785x; 3.3785x over previous
import functools

import jax
import jax.numpy as jnp
from jax import lax
from jax.experimental import pallas as pl
from jax.experimental.pallas import tpu as pltpu
from jax.experimental.pallas import tpu_sc as plsc

VOCAB = 100000
DIM = 300
BATCH = 4096
BAG = 50

NUM_CORES = 2
NUM_SUBCORES = 16
NW = NUM_CORES * NUM_SUBCORES
BPW = BATCH // NW
LANES = 16
TILE = 128
TAIL = DIM - 2 * TILE
SCALE = 1.0 / BAG

_FULL_STARTS = [16 * i for i in range(TILE // 16)]
_TAIL_STARTS = [0, 16, 32]
DIM_PAD = 3 * TILE
_CHUNKS = (
    [(0, s) for s in _FULL_STARTS]
    + [(1, s) for s in _FULL_STARTS]
    + [(2, s) for s in _TAIL_STARTS]
)
NCHUNK = len(_CHUNKS)

_mesh = plsc.VectorSubcoreMesh(core_axis_name="c", subcore_axis_name="s")


@functools.partial(
    pl.kernel,
    mesh=_mesh,
    out_type=jax.ShapeDtypeStruct((BATCH, DIM_PAD), jnp.float32),
    scratch_types=[
        pltpu.VMEM((BPW, BAG), jnp.int32),
        pltpu.VMEM((2, 3, BAG, TILE), jnp.float32),
        pltpu.VMEM((BPW, DIM_PAD), jnp.float32),
        pltpu.SemaphoreType.DMA,
        pltpu.SemaphoreType.DMA,
    ],
)
def _bow_sc(idx_hbm, table_hbm, tail_hbm, out_hbm, idx_v, rows_v, out_v,
            sem0, sem1):
    wid = lax.axis_index("s") * NUM_CORES + lax.axis_index("c")
    base = wid * BPW
    sems = (sem0, sem1)

    pltpu.sync_copy(idx_hbm.at[pl.ds(base, BPW)], idx_v)

    def srcs(g):
        idx = idx_v.at[g]
        return (
            table_hbm.at[idx, pl.ds(0, TILE)],
            table_hbm.at[idx, pl.ds(TILE, TILE)],
            tail_hbm.at[idx],
        )

    def issue(g, buf):
        for j, src in enumerate(srcs(g)):
            pltpu.async_copy(src, rows_v.at[buf, j], sems[buf])

    def wait_buf(g, buf):
        for j, src in enumerate(srcs(g)):
            pltpu.make_async_copy(src, rows_v.at[buf, j], sems[buf]).wait()

    def reduce_bag(g, buf):
        def body(r, accs):
            return tuple(
                accs[i] + rows_v[buf, j, r, pl.ds(s, LANES)]
                for i, (j, s) in enumerate(_CHUNKS)
            )

        zero = jnp.zeros((LANES,), jnp.float32)
        accs = lax.fori_loop(0, BAG, body, (zero,) * NCHUNK)
        for i, (j, s) in enumerate(_CHUNKS):
            out_v[g, pl.ds(j * TILE + s, LANES)] = accs[i] * SCALE

    issue(0, 0)

    def pair_body(p, carry):
        for h in range(2):
            g = p * 2 + h

            @pl.when(g + 1 < BPW)
            def _():
                issue(g + 1, 1 - h)

            wait_buf(g, h)
            reduce_bag(g, h)
        return carry

    lax.fori_loop(0, BPW // 2, pair_body, 0)
    pltpu.sync_copy(out_v, out_hbm.at[pl.ds(base, BPW)])


def kernel(indices, table):
    idx = jnp.asarray(indices, jnp.int32)
    tail = jnp.pad(table[:, 2 * TILE:], ((0, 0), (0, TILE - TAIL)))
    return _bow_sc(idx, table, tail)[:, :DIM]
